# baseline (speedup 1.0000x reference)
import functools

import jax
import jax.numpy as jnp
from jax import lax
from jax.experimental import pallas as pl
from jax.experimental.pallas import tpu as pltpu
from jax.experimental.pallas import tpu_sc as plsc

_B = 16384
_D = 64
_L = 16

_NC, _NS = 2, 16
_NW = _NC * _NS
_BPW = _B // _NW


_GATHER_DNUMS = lax.GatherDimensionNumbers(
    offset_dims=(), collapsed_slice_dims=(0,), start_index_map=(0,)
)


def _lane_permute(v, idx):
    return lax.gather(
        v,
        idx[:, None],
        _GATHER_DNUMS,
        slice_sizes=(1,),
        mode=lax.GatherScatterMode.PROMISE_IN_BOUNDS,
    )


def _lanes_reduce(v, op):
    lanes = lax.iota(jnp.int32, _L)
    for k in (8, 4, 2, 1):
        v = op(v, _lane_permute(v, lanes ^ k))
    return v


def _sc_body(table_hbm, idx_hbm, out_hbm, idx_v, rows_v, sem):
    wid = lax.axis_index("s") * _NC + lax.axis_index("c")
    base = wid * _BPW
    pltpu.sync_copy(idx_hbm.at[pl.ds(base, _BPW)], idx_v)
    pltpu.async_copy(table_hbm.at[idx_v], rows_v, sem).wait()

    def row(r, carry):
        v0 = rows_v[r, pl.ds(0, _L)]
        v1 = rows_v[r, pl.ds(_L, _L)]
        v2 = rows_v[r, pl.ds(2 * _L, _L)]
        v3 = rows_v[r, pl.ds(3 * _L, _L)]
        m = _lanes_reduce(
            jnp.maximum(jnp.maximum(v0, v1), jnp.maximum(v2, v3)), jnp.maximum
        )
        e0 = jnp.exp(v0 - m)
        e1 = jnp.exp(v1 - m)
        e2 = jnp.exp(v2 - m)
        e3 = jnp.exp(v3 - m)
        inv = 1.0 / _lanes_reduce((e0 + e1) + (e2 + e3), jnp.add)
        rows_v[r, pl.ds(0, _L)] = e0 * inv
        rows_v[r, pl.ds(_L, _L)] = e1 * inv
        rows_v[r, pl.ds(2 * _L, _L)] = e2 * inv
        rows_v[r, pl.ds(3 * _L, _L)] = e3 * inv
        return carry

    lax.fori_loop(0, _BPW, row, 0)
    pltpu.sync_copy(rows_v, out_hbm.at[pl.ds(base, _BPW)])


@jax.jit
def kernel(logits, s_idx):
    f = functools.partial(
        pl.kernel,
        mesh=plsc.VectorSubcoreMesh(core_axis_name="c", subcore_axis_name="s"),
        out_type=jax.ShapeDtypeStruct((_B, _D), jnp.float32),
        scratch_types=[
            pltpu.VMEM((_BPW,), jnp.int32),
            pltpu.VMEM((_BPW, _D), jnp.float32),
            pltpu.SemaphoreType.DMA,
        ],
        compiler_params=pltpu.CompilerParams(use_tc_tiling_on_sc=False),
    )(_sc_body)
    return f(logits, s_idx)


# speedup vs baseline: 1.0014x; 1.0014x over previous
import functools

import jax
import jax.numpy as jnp
from jax import lax
from jax.experimental import pallas as pl
from jax.experimental.pallas import tpu as pltpu
from jax.experimental.pallas import tpu_sc as plsc

_S = 1000000
_B = 16384
_D = 64
_L = 16

_NC, _NS = 2, 16
_NW = _NC * _NS
_BPW = _B // _NW

_GATHER_DNUMS = lax.GatherDimensionNumbers(
    offset_dims=(), collapsed_slice_dims=(0,), start_index_map=(0,)
)


def _lane_permute(v, idx):
    return lax.gather(
        v,
        idx[:, None],
        _GATHER_DNUMS,
        slice_sizes=(1,),
        mode=lax.GatherScatterMode.PROMISE_IN_BOUNDS,
    )


def _lanes_reduce(v, op, lanes):
    for k in (8, 4, 2, 1):
        v = op(v, _lane_permute(v, lanes ^ k))
    return v


def _sc_body(tbl_hbm, idx_hbm, out_hbm, idx_v, widx_v, rows_v, out_v, sem):
    wid = lax.axis_index("s") * _NC + lax.axis_index("c")
    base = wid * _BPW
    pltpu.sync_copy(idx_hbm.at[pl.ds(base, _BPW)], idx_v)

    def halve(g, _):
        iv = idx_v[pl.ds(g * _L, _L)]
        widx_v[pl.ds(g * _L, _L)] = lax.shift_right_logical(iv, 1)
        return 0

    lax.fori_loop(0, _BPW // _L, halve, 0)

    lanes = lax.iota(jnp.int32, _L)
    _HB = _BPW // 2

    for half in range(2):
        hb = half * _HB
        pltpu.async_copy(
            tbl_hbm.at[widx_v.at[pl.ds(hb, _HB)]], rows_v, sem
        ).wait()

        def group(g, _):
            gb = g * _L
            iv16 = idx_v[pl.ds(hb + gb, _L)]
            for e in range(_L):
                r = gb + e
                h = (iv16[e] & 1) * _D
                v0 = rows_v[r, pl.ds(h, _L)]
                v1 = rows_v[r, pl.ds(h + _L, _L)]
                v2 = rows_v[r, pl.ds(h + 2 * _L, _L)]
                v3 = rows_v[r, pl.ds(h + 3 * _L, _L)]
                m = _lanes_reduce(
                    jnp.maximum(jnp.maximum(v0, v1), jnp.maximum(v2, v3)),
                    jnp.maximum,
                    lanes,
                )
                e0 = jnp.exp(v0 - m)
                e1 = jnp.exp(v1 - m)
                e2 = jnp.exp(v2 - m)
                e3 = jnp.exp(v3 - m)
                inv = 1.0 / _lanes_reduce(
                    (e0 + e1) + (e2 + e3), jnp.add, lanes
                )
                out_v[r, pl.ds(0, _L)] = e0 * inv
                out_v[r, pl.ds(_L, _L)] = e1 * inv
                out_v[r, pl.ds(2 * _L, _L)] = e2 * inv
                out_v[r, pl.ds(3 * _L, _L)] = e3 * inv
            return 0

        lax.fori_loop(0, _HB // _L, group, 0)

        pltpu.sync_copy(out_v, out_hbm.at[pl.ds(base + hb, _HB)])


@jax.jit
def kernel(logits, s_idx):
    f = functools.partial(
        pl.kernel,
        mesh=plsc.VectorSubcoreMesh(core_axis_name="c", subcore_axis_name="s"),
        out_type=jax.ShapeDtypeStruct((_B, _D), jnp.float32),
        scratch_types=[
            pltpu.VMEM((_BPW,), jnp.int32),
            pltpu.VMEM((_BPW,), jnp.int32),
            pltpu.VMEM((_BPW // 2, 2 * _D), jnp.float32),
            pltpu.VMEM((_BPW // 2, _D), jnp.float32),
            pltpu.SemaphoreType.DMA,
        ],
        compiler_params=pltpu.CompilerParams(needs_layout_passes=False),
    )(_sc_body)
    return f(logits.reshape(_S // 2, 2 * _D), s_idx)


# speedup vs baseline: 1.1366x; 1.1349x over previous
import functools

import jax
import jax.numpy as jnp
from jax import lax
from jax.experimental import pallas as pl
from jax.experimental.pallas import tpu as pltpu
from jax.experimental.pallas import tpu_sc as plsc

_S = 1000000
_B = 16384
_D = 64
_L = 16

_NC, _NS = 2, 16
_NW = _NC * _NS
_BPW = _B // _NW

_GATHER_DNUMS = lax.GatherDimensionNumbers(
    offset_dims=(), collapsed_slice_dims=(0,), start_index_map=(0,)
)


def _lane_permute(v, idx):
    return lax.gather(
        v,
        idx[:, None],
        _GATHER_DNUMS,
        slice_sizes=(1,),
        mode=lax.GatherScatterMode.PROMISE_IN_BOUNDS,
    )


def _lanes_reduce(v, op, lanes):
    for k in (8, 4, 2, 1):
        v = op(v, _lane_permute(v, lanes ^ k))
    return v


def _sc_body(tbl_hbm, idx_hbm, out_hbm, idx_v, rows_v, out_v, sem):
    wid = lax.axis_index("s") * _NC + lax.axis_index("c")
    base = wid * _BPW
    pltpu.sync_copy(idx_hbm.at[pl.ds(base, _BPW)], idx_v)

    lanes = lax.iota(jnp.int32, _L)
    _HB = _BPW // 2

    for half in range(2):
        hb = half * _HB
        pltpu.async_copy(
            tbl_hbm.at[idx_v.at[pl.ds(hb, _HB)]], rows_v, sem
        ).wait()

        def group(g, _):
            gb = g * _L
            for e in range(_L):
                r = gb + e
                v0 = rows_v[r, pl.ds(0, _L)]
                v1 = rows_v[r, pl.ds(_L, _L)]
                v2 = rows_v[r, pl.ds(2 * _L, _L)]
                v3 = rows_v[r, pl.ds(3 * _L, _L)]
                m = _lanes_reduce(
                    jnp.maximum(jnp.maximum(v0, v1), jnp.maximum(v2, v3)),
                    jnp.maximum,
                    lanes,
                )
                e0 = jnp.exp(v0 - m)
                e1 = jnp.exp(v1 - m)
                e2 = jnp.exp(v2 - m)
                e3 = jnp.exp(v3 - m)
                inv = 1.0 / _lanes_reduce(
                    (e0 + e1) + (e2 + e3), jnp.add, lanes
                )
                out_v[r, pl.ds(0, _L)] = e0 * inv
                out_v[r, pl.ds(_L, _L)] = e1 * inv
                out_v[r, pl.ds(2 * _L, _L)] = e2 * inv
                out_v[r, pl.ds(3 * _L, _L)] = e3 * inv
            return 0

        lax.fori_loop(0, _HB // _L, group, 0)

        pltpu.sync_copy(out_v, out_hbm.at[pl.ds(base + hb, _HB)])


@jax.jit
def kernel(logits, s_idx):
    f = functools.partial(
        pl.kernel,
        mesh=plsc.VectorSubcoreMesh(core_axis_name="c", subcore_axis_name="s"),
        out_type=jax.ShapeDtypeStruct((_B, _D), jnp.float32),
        scratch_types=[
            pltpu.VMEM((_BPW,), jnp.int32),
            pltpu.VMEM((_BPW // 2, 2 * _D), jnp.float32),
            pltpu.VMEM((_BPW // 2, _D), jnp.float32),
            pltpu.SemaphoreType.DMA,
        ],
        compiler_params=pltpu.CompilerParams(needs_layout_passes=False),
    )(_sc_body)
    return f(jnp.pad(logits, ((0, 0), (0, _D))), s_idx)
